# baseline (speedup 1.0000x reference)
import functools

import jax
import jax.numpy as jnp
from jax import lax
from jax.experimental import pallas as pl
from jax.experimental.pallas import tpu as pltpu
from jax.experimental.pallas import tpu_sc as plsc

B, L, V = 4096, 200, 1000
LANES = 16
NC, NS = 2, 16
NW = NC * NS
RPW = B // NW
GROUPS = RPW // LANES
NBUF = 4
UNROLL = 8


def _tfidf_body(tok_hbm, idf_hbm, out_hbm, tok_v, idf_v, hist,
                sem0, sem1, sem2, sem3, semt):
    wid = lax.axis_index("s") * NC + lax.axis_index("c")
    base = wid * RPW

    tok_cp = pltpu.make_async_copy(
        tok_hbm.at[pl.ds(base, RPW), :], tok_v, semt
    )
    tok_cp.start()
    pltpu.sync_copy(idf_hbm, idf_v)

    lanes = lax.iota(jnp.int32, 16)
    zf = jnp.zeros((16,), jnp.float32)
    sems = (sem0, sem1, sem2, sem3)

    def out_copy(g):
        return pltpu.make_async_copy(
            hist.at[g % NBUF],
            out_hbm.at[pl.ds(base + g * LANES, LANES), :],
            sems[g % NBUF],
        )

    def zbody(i, _):
        off = i * 16
        for b in range(NBUF):
            for r in range(LANES):
                hist[b, r, pl.ds(off, 16)] = zf
        return _

    lax.fori_loop(0, V // 16, zbody, None)
    for b in range(NBUF):
        for r in range(LANES):
            hist[b, r, pl.ds(V - 16, 16)] = zf

    tok_cp.wait()

    for g in range(GROUPS):
        h = hist.at[g % NBUF]
        rows = lanes + g * LANES

        if g >= 2:
            out_copy(g - 2).wait()

        if 2 <= g < GROUPS - 2:
            h_old = hist.at[(g - 2) % NBUF]
            rows_old = lanes + (g - 2) * LANES

            @plsc.parallel_loop(0, L, unroll=UNROLL)
            def _(l, _h=h, _rows=rows, _h_old=h_old, _rows_old=rows_old):
                lv = jnp.full((16,), l, dtype=jnp.int32)
                tok = plsc.load_gather(tok_v, [_rows, lv])
                w = plsc.load_gather(idf_v, [tok])
                plsc.addupdate_scatter(_h, [lanes, tok], w)
                tok_old = plsc.load_gather(tok_v, [_rows_old, lv])
                plsc.store_scatter(_h_old, [lanes, tok_old], zf)
        else:
            @plsc.parallel_loop(0, L, unroll=UNROLL)
            def _(l, _h=h, _rows=rows):
                lv = jnp.full((16,), l, dtype=jnp.int32)
                tok = plsc.load_gather(tok_v, [_rows, lv])
                w = plsc.load_gather(idf_v, [tok])
                plsc.addupdate_scatter(_h, [lanes, tok], w)

        out_copy(g).start()

    out_copy(GROUPS - 2).wait()
    out_copy(GROUPS - 1).wait()


_tfidf = functools.partial(
    pl.kernel,
    out_type=jax.ShapeDtypeStruct((B, V), jnp.float32),
    mesh=plsc.VectorSubcoreMesh(core_axis_name="c", subcore_axis_name="s"),
    compiler_params=pltpu.CompilerParams(
        use_tc_tiling_on_sc=True, needs_layout_passes=False
    ),
    scratch_types=[
        pltpu.VMEM((RPW, L), jnp.int32),
        pltpu.VMEM((V,), jnp.float32),
        pltpu.VMEM((NBUF, LANES, V), jnp.float32),
        pltpu.SemaphoreType.DMA,
        pltpu.SemaphoreType.DMA,
        pltpu.SemaphoreType.DMA,
        pltpu.SemaphoreType.DMA,
        pltpu.SemaphoreType.DMA,
    ],
)(_tfidf_body)


def kernel(token_ids, idf_weights):
    return _tfidf(token_ids, idf_weights)


# speedup vs baseline: 1.2544x; 1.2544x over previous
import functools

import jax
import jax.numpy as jnp
from jax import lax
from jax.experimental import pallas as pl
from jax.experimental.pallas import tpu as pltpu
from jax.experimental.pallas import tpu_sc as plsc

B, L, V = 4096, 200, 1000
LANES = 16
NC, NS = 2, 16
NW = NC * NS
RPW = B // NW
GROUPS = RPW // LANES
NBUF = 3
UNROLL = 8

_CHUNKS = (0, 16, 32, 48, 64, 80, 96, 112, 128, 144, 160, 176, 184)


def _tfidf_body(tok_hbm, idf_hbm, out_hbm, tok_t, tok_v, idf_v, hist,
                sem0, sem1, sem2, semt):
    wid = lax.axis_index("s") * NC + lax.axis_index("c")
    base = wid * RPW

    tok_cp = pltpu.make_async_copy(
        tok_hbm.at[pl.ds(base, RPW), :], tok_t, semt
    )
    tok_cp.start()
    pltpu.sync_copy(idf_hbm, idf_v)

    lanes = lax.iota(jnp.int32, 16)
    zf = jnp.zeros((16,), jnp.float32)
    sems = (sem0, sem1, sem2)

    def out_copy(g):
        return pltpu.make_async_copy(
            hist.at[g % NBUF],
            out_hbm.at[pl.ds(base + g * LANES, LANES), :],
            sems[g % NBUF],
        )

    def zbody(i, _):
        off = i * 16
        for b in range(NBUF):
            for r in range(LANES):
                hist[b, r, pl.ds(off, 16)] = zf
        return _

    lax.fori_loop(0, V // 16, zbody, None)
    for b in range(NBUF):
        for r in range(LANES):
            hist[b, r, pl.ds(V - 16, 16)] = zf

    tok_cp.wait()

    @plsc.parallel_loop(0, RPW, unroll=2)
    def _(r):
        for c in _CHUNKS:
            tok_v[pl.ds(r * L + c, 16)] = tok_t[r, pl.ds(c, 16)]

    for g in range(GROUPS):
        h = hist.at[g % NBUF]
        rows = (lanes + g * LANES) * L

        if g >= 2:
            out_copy(g - 2).wait()

        if 2 <= g < GROUPS - 1:
            h_old = hist.at[(g - 2) % NBUF]
            rows_old = (lanes + (g - 2) * LANES) * L

            @plsc.parallel_loop(0, L, unroll=UNROLL)
            def _(l, _h=h, _rows=rows, _h_old=h_old, _rows_old=rows_old):
                lv = jnp.full((16,), l, dtype=jnp.int32)
                tok = plsc.load_gather(tok_v, [_rows + lv])
                w = plsc.load_gather(idf_v, [tok])
                plsc.addupdate_scatter(_h, [lanes, tok], w)
                tok_old = plsc.load_gather(tok_v, [_rows_old + lv])
                plsc.store_scatter(_h_old, [lanes, tok_old], zf)
        else:
            @plsc.parallel_loop(0, L, unroll=UNROLL)
            def _(l, _h=h, _rows=rows):
                lv = jnp.full((16,), l, dtype=jnp.int32)
                tok = plsc.load_gather(tok_v, [_rows + lv])
                w = plsc.load_gather(idf_v, [tok])
                plsc.addupdate_scatter(_h, [lanes, tok], w)

        out_copy(g).start()

    out_copy(GROUPS - 2).wait()
    out_copy(GROUPS - 1).wait()


_tfidf = functools.partial(
    pl.kernel,
    out_type=jax.ShapeDtypeStruct((B, V), jnp.float32),
    mesh=plsc.VectorSubcoreMesh(core_axis_name="c", subcore_axis_name="s"),
    compiler_params=pltpu.CompilerParams(
        use_tc_tiling_on_sc=True, needs_layout_passes=False
    ),
    scratch_types=[
        pltpu.VMEM((RPW, L), jnp.int32),
        pltpu.VMEM((RPW * L,), jnp.int32),
        pltpu.VMEM((V,), jnp.float32),
        pltpu.VMEM((NBUF, LANES, V), jnp.float32),
        pltpu.SemaphoreType.DMA,
        pltpu.SemaphoreType.DMA,
        pltpu.SemaphoreType.DMA,
        pltpu.SemaphoreType.DMA,
    ],
)(_tfidf_body)


def kernel(token_ids, idf_weights):
    return _tfidf(token_ids, idf_weights)


# speedup vs baseline: 1.2570x; 1.0021x over previous
import functools

import jax
import jax.numpy as jnp
from jax import lax
from jax.experimental import pallas as pl
from jax.experimental.pallas import tpu as pltpu
from jax.experimental.pallas import tpu_sc as plsc

B, L, V = 4096, 200, 1000
LANES = 16
NC, NS = 2, 16
NW = NC * NS
RPW = B // NW
GROUPS = RPW // LANES
NBUF = 3
UNROLL = 8

_CHUNKS = (0, 16, 32, 48, 64, 80, 96, 112, 128, 144, 160, 176, 184)


def _tfidf_body(tok_hbm, idf_hbm, out_hbm, tok_t, tok_v, idf_v, hist,
                sem0, sem1, sem2, semt):
    wid = lax.axis_index("s") * NC + lax.axis_index("c")
    base = wid * RPW

    tok_cp = pltpu.make_async_copy(
        tok_hbm.at[pl.ds(base, RPW), :], tok_t, semt
    )
    tok_cp.start()
    pltpu.sync_copy(idf_hbm, idf_v)

    lanes = lax.iota(jnp.int32, 16)
    zf = jnp.zeros((16,), jnp.float32)
    sems = (sem0, sem1, sem2)

    def out_copy(g):
        return pltpu.make_async_copy(
            hist.at[g % NBUF],
            out_hbm.at[pl.ds(base + g * LANES, LANES), :],
            sems[g % NBUF],
        )

    def zbody(i, _):
        off = i * 16
        for b in range(NBUF):
            for r in range(LANES):
                hist[b, r, pl.ds(off, 16)] = zf
        return _

    lax.fori_loop(0, V // 16, zbody, None)
    for b in range(NBUF):
        for r in range(LANES):
            hist[b, r, pl.ds(V - 16, 16)] = zf

    tok_cp.wait()

    @plsc.parallel_loop(0, RPW, unroll=4)
    def _(r):
        for c in _CHUNKS:
            tok_v[pl.ds(r * L + c, 16)] = tok_t[r, pl.ds(c, 16)]

    for g in range(GROUPS):
        h = hist.at[g % NBUF]
        rows = (lanes + g * LANES) * L

        if g >= 2:
            out_copy(g - 2).wait()

        if 2 <= g < GROUPS - 1:
            h_old = hist.at[(g - 2) % NBUF]
            rows_old = (lanes + (g - 2) * LANES) * L

            @plsc.parallel_loop(0, L, unroll=UNROLL)
            def _(l, _h=h, _rows=rows, _h_old=h_old, _rows_old=rows_old):
                lv = jnp.full((16,), l, dtype=jnp.int32)
                tok = plsc.load_gather(tok_v, [_rows + lv])
                w = plsc.load_gather(idf_v, [tok])
                plsc.addupdate_scatter(_h, [lanes, tok], w)
                tok_old = plsc.load_gather(tok_v, [_rows_old + lv])
                plsc.store_scatter(_h_old, [lanes, tok_old], zf)
        else:
            @plsc.parallel_loop(0, L, unroll=UNROLL)
            def _(l, _h=h, _rows=rows):
                lv = jnp.full((16,), l, dtype=jnp.int32)
                tok = plsc.load_gather(tok_v, [_rows + lv])
                w = plsc.load_gather(idf_v, [tok])
                plsc.addupdate_scatter(_h, [lanes, tok], w)

        out_copy(g).start()

    out_copy(GROUPS - 2).wait()
    out_copy(GROUPS - 1).wait()


_tfidf = functools.partial(
    pl.kernel,
    out_type=jax.ShapeDtypeStruct((B, V), jnp.float32),
    mesh=plsc.VectorSubcoreMesh(core_axis_name="c", subcore_axis_name="s"),
    compiler_params=pltpu.CompilerParams(
        use_tc_tiling_on_sc=True, needs_layout_passes=False
    ),
    scratch_types=[
        pltpu.VMEM((RPW, L), jnp.int32),
        pltpu.VMEM((RPW * L,), jnp.int32),
        pltpu.VMEM((V,), jnp.float32),
        pltpu.VMEM((NBUF, LANES, V), jnp.float32),
        pltpu.SemaphoreType.DMA,
        pltpu.SemaphoreType.DMA,
        pltpu.SemaphoreType.DMA,
        pltpu.SemaphoreType.DMA,
    ],
)(_tfidf_body)


def kernel(token_ids, idf_weights):
    return _tfidf(token_ids, idf_weights)
